# baseline (speedup 1.0000x reference)
import functools

import jax
import jax.numpy as jnp
from jax import lax
from jax.experimental import pallas as pl
from jax.experimental.pallas import tpu as pltpu
from jax.experimental.pallas import tpu_sc as plsc

DIM = 768
NW = 32
CH = 40


def _sc_gather(table, idx3, batch):
    bpw = batch // NW
    nch = bpw // CH
    num_emb = table.shape[0]
    mesh = plsc.VectorSubcoreMesh(core_axis_name="c", subcore_axis_name="s")

    @functools.partial(
        pl.kernel,
        out_type=jax.ShapeDtypeStruct((batch, DIM), jnp.float32),
        mesh=mesh,
        scratch_types=[
            pltpu.VMEM((nch, CH), jnp.int32),
            pltpu.VMEM((2, CH, DIM), jnp.float32),
            pltpu.SemaphoreType.DMA,
            pltpu.SemaphoreType.DMA,
        ],
    )
    def k(table_hbm, idx_hbm, out_hbm, idx_v, rows_v, gsem, osem):
        wid = lax.axis_index("s") * 2 + lax.axis_index("c")
        base = wid * bpw
        pltpu.sync_copy(idx_hbm.at[wid], idx_v)

        def gather(c, slot):
            return pltpu.make_async_copy(
                table_hbm.at[idx_v.at[c]], rows_v.at[slot], gsem
            )

        def write(c, slot):
            return pltpu.make_async_copy(
                rows_v.at[slot], out_hbm.at[pl.ds(base + c * CH, CH)], osem
            )

        gather(0, 0).start()

        def body(c, _):
            slot = lax.rem(c, 2)
            nslot = lax.rem(c + 1, 2)
            gather(c, slot).wait()
            write(c, slot).start()

            @pl.when(c >= 1)
            def _():
                write(c - 1, nslot).wait()

            @pl.when(c + 1 < nch)
            def _():
                gather(c + 1, nslot).start()

            return 0

        lax.fori_loop(0, nch, body, 0, unroll=False)
        write(nch - 1, lax.rem(nch - 1, 2)).wait()

    return k(table, idx3)


def kernel(x, table):
    batch = x.shape[0] * x.shape[1]
    idx = x.reshape(-1).astype(jnp.int32)
    idx3 = idx.reshape(NW, batch // (NW * CH), CH)
    return _sc_gather(table, idx3, batch)


# speedup vs baseline: 1.0348x; 1.0348x over previous
import functools

import jax
import jax.numpy as jnp
from jax import lax
from jax.experimental import pallas as pl
from jax.experimental.pallas import tpu as pltpu
from jax.experimental.pallas import tpu_sc as plsc

DIM = 768
NW = 32
CH = 40


def _sc_gather(table, idx3, batch):
    bpw = batch // NW
    nch = bpw // CH
    num_emb = table.shape[0]
    mesh = plsc.VectorSubcoreMesh(core_axis_name="c", subcore_axis_name="s")

    @functools.partial(
        pl.kernel,
        out_type=jax.ShapeDtypeStruct((batch, DIM), jnp.float32),
        mesh=mesh,
        scratch_types=[
            pltpu.VMEM((nch, CH), jnp.int32),
            pltpu.VMEM((3, CH, DIM), jnp.float32),
            pltpu.SemaphoreType.DMA,
            pltpu.SemaphoreType.DMA,
        ],
    )
    def k(table_hbm, idx_hbm, out_hbm, idx_v, rows_v, gsem, osem):
        wid = lax.axis_index("s") * 2 + lax.axis_index("c")
        base = wid * bpw
        pltpu.sync_copy(idx_hbm.at[wid], idx_v)

        def gather(c, slot):
            return pltpu.make_async_copy(
                table_hbm.at[idx_v.at[c]], rows_v.at[slot], gsem
            )

        def write(c, slot):
            return pltpu.make_async_copy(
                rows_v.at[slot], out_hbm.at[pl.ds(base + c * CH, CH)], osem
            )

        gather(0, 0).start()
        gather(1, 1).start()

        def body(c, _):
            slot = lax.rem(c, 3)
            gather(c, slot).wait()
            write(c, slot).start()

            @pl.when(c >= 1)
            def _():
                write(c - 1, lax.rem(c - 1, 3)).wait()

            @pl.when(c + 2 < nch)
            def _():
                gather(c + 2, lax.rem(c + 2, 3)).start()

            return 0

        lax.fori_loop(0, nch, body, 0, unroll=False)
        write(nch - 1, lax.rem(nch - 1, 3)).wait()

    return k(table, idx3)


def kernel(x, table):
    batch = x.shape[0] * x.shape[1]
    idx = x.reshape(-1).astype(jnp.int32)
    idx3 = idx.reshape(NW, batch // (NW * CH), CH)
    return _sc_gather(table, idx3, batch)
